# Initial kernel scaffold; baseline (speedup 1.0000x reference)
import jax
import jax.numpy as jnp
from jax.experimental import pallas as pl


def kernel(x, y):
    raise NotImplementedError("write your pallas kernel here")


# speedup vs baseline: 3.7706x; 3.7706x over previous
import functools

import jax
import jax.numpy as jnp
from jax.experimental import pallas as pl
from jax.experimental.pallas import tpu as pltpu

_K = 5
_P = 3
_S = 1

_Q = 2116
_D = 2304
_QB = 128
_NBLK = 17
_QPAD = _QB * _NBLK
_NEG = -3.0e38


def _extract_patches(feat, p, s):
    patches = jax.lax.conv_general_dilated_patches(
        feat, filter_shape=(p, p), window_strides=(s, s), padding='VALID')
    B, D, nh, nw = patches.shape
    return patches.reshape(B, D, nh * nw).transpose(0, 2, 1)


def _topk_body(q_ref, kt_ref, sc_ref, ix_ref, kb16_ref, k2_ref):
    i = pl.program_id(0)

    @pl.when(i == 0)
    def _():
        kt = kt_ref[...]
        kb16_ref[...] = kt.astype(jnp.bfloat16)
        k2_ref[0:1, :] = jnp.sum(kt * kt, axis=0, keepdims=True)

    qb = q_ref[...]
    q2 = jnp.sum(qb * qb, axis=1)
    qk = jax.lax.dot_general(
        qb.astype(jnp.bfloat16), kb16_ref[...],
        dimension_numbers=(((1,), (0,)), ((), ())),
        preferred_element_type=jnp.float32)
    t = 2.0 * qk - k2_ref[0:1, :]
    col = jax.lax.broadcasted_iota(jnp.int32, (_QB, _QPAD), 1)
    t = jnp.where(col < _Q, t, _NEG)
    for kk in range(_K):
        m = jnp.max(t, axis=1)
        a = jnp.argmax(t, axis=1).astype(jnp.int32)
        sc_ref[:, kk] = m - q2
        ix_ref[:, kk] = a
        t = jnp.where(col == a[:, None], _NEG, t)


def _diff_body(idx_sm, q_ref, kp_ref, o_ref):
    i = pl.program_id(0)

    def row(r, _):
        qrow = q_ref[pl.ds(r, 1), :]
        for kk in range(_K):
            n = idx_sm[i * _QB * _K + r * _K + kk]
            o_ref[pl.ds(r * _K + kk, 1), :] = kp_ref[pl.ds(n, 1), :] - qrow
        return 0

    jax.lax.fori_loop(0, _QB, row, 0)


def kernel(x, y):
    q = _extract_patches(x, _P, _S)[0]
    kp = _extract_patches(y, _P, _S)[0]
    pad = _QPAD - _Q
    qp = jnp.pad(q, ((0, pad), (0, 0)))
    kpp = jnp.pad(kp, ((0, pad), (0, 0)))
    kpt = kpp.T

    scores, idxs = pl.pallas_call(
        _topk_body,
        grid=(_NBLK,),
        in_specs=[
            pl.BlockSpec((_QB, _D), lambda i: (i, 0)),
            pl.BlockSpec((_D, _QPAD), lambda i: (0, 0)),
        ],
        out_specs=[
            pl.BlockSpec((_QB, 8), lambda i: (i, 0)),
            pl.BlockSpec((_QB, 8), lambda i: (i, 0)),
        ],
        out_shape=[
            jax.ShapeDtypeStruct((_QPAD, 8), jnp.float32),
            jax.ShapeDtypeStruct((_QPAD, 8), jnp.int32),
        ],
        scratch_shapes=[
            pltpu.VMEM((_D, _QPAD), jnp.bfloat16),
            pltpu.VMEM((8, _QPAD), jnp.float32),
        ],
    )(qp, kpt)

    idx_flat = idxs[:, :_K].reshape(_QPAD * _K)

    diff_flat = pl.pallas_call(
        _diff_body,
        grid_spec=pltpu.PrefetchScalarGridSpec(
            num_scalar_prefetch=1,
            grid=(_NBLK,),
            in_specs=[
                pl.BlockSpec((_QB, _D), lambda i, _: (i, 0)),
                pl.BlockSpec((_QPAD, _D), lambda i, _: (0, 0)),
            ],
            out_specs=pl.BlockSpec((_QB * _K, _D), lambda i, _: (i, 0)),
        ),
        out_shape=jax.ShapeDtypeStruct((_QPAD * _K, _D), jnp.float32),
    )(idx_flat, qp, kpp)

    score_k = scores[:_Q, :_K][None]
    idx_k = idxs[:_Q, :_K][None]
    diff_patch = diff_flat.reshape(_QPAD, _K, _D)[:_Q][None]
    return (score_k, idx_k, diff_patch)


# speedup vs baseline: 4.2592x; 1.1296x over previous
import functools

import jax
import jax.numpy as jnp
from jax.experimental import pallas as pl
from jax.experimental.pallas import tpu as pltpu
from jax.experimental.pallas import tpu_sc as plsc

_K = 5
_P = 3
_S = 1

_Q = 2116
_D = 2304
_QB = 128
_NBLK = 17
_QPAD = _QB * _NBLK
_NEG = -3.0e38


def _extract_patches(feat, p, s):
    patches = jax.lax.conv_general_dilated_patches(
        feat, filter_shape=(p, p), window_strides=(s, s), padding='VALID')
    B, D, nh, nw = patches.shape
    return patches.reshape(B, D, nh * nw).transpose(0, 2, 1)


def _topk_body(q_ref, kt_ref, sc_ref, ix_ref, kb16_ref, k2_ref):
    i = pl.program_id(0)

    @pl.when(i == 0)
    def _():
        kt = kt_ref[...]
        kb16_ref[...] = kt.astype(jnp.bfloat16)
        k2_ref[0:1, :] = jnp.sum(kt * kt, axis=0, keepdims=True)

    qb = q_ref[...]
    q2 = jnp.sum(qb * qb, axis=1)
    qk = jax.lax.dot_general(
        qb.astype(jnp.bfloat16), kb16_ref[...],
        dimension_numbers=(((1,), (0,)), ((), ())),
        preferred_element_type=jnp.float32)
    t = 2.0 * qk - k2_ref[0:1, :]
    col = jax.lax.broadcasted_iota(jnp.int32, (_QB, _QPAD), 1)
    t = jnp.where(col < _Q, t, _NEG)
    for kk in range(_K):
        m = jnp.max(t, axis=1)
        a = jnp.argmax(t, axis=1).astype(jnp.int32)
        sc_ref[:, kk] = m - q2
        ix_ref[:, kk] = a
        t = jnp.where(col == a[:, None], _NEG, t)


_NW = 32
_TPW = 352
_TPAD = _NW * _TPW
_W = 16
_NWIN = _TPW // _W


def _sc_gather(kpp, idx_flat, tidx_flat):
    mesh = plsc.VectorSubcoreMesh(core_axis_name="c", subcore_axis_name="s")

    @functools.partial(
        pl.kernel, mesh=mesh,
        out_type=jax.ShapeDtypeStruct((_QPAD * 8, _D), jnp.float32),
        scratch_types=[
            pltpu.VMEM((_W,), jnp.int32),
            pltpu.VMEM((_W,), jnp.int32),
            pltpu.VMEM((_W, _D), jnp.float32),
            pltpu.SemaphoreType.DMA,
            pltpu.SemaphoreType.DMA,
        ],
    )
    def body(kp_hbm, idx_hbm, tidx_hbm, out_hbm, idxw, tidxw, rows, s1, s2):
        wid = jax.lax.axis_index("s") * 2 + jax.lax.axis_index("c")
        base = wid * _TPW

        @pl.loop(0, _NWIN)
        def _(w):
            b = base + w * _W
            pltpu.sync_copy(idx_hbm.at[pl.ds(b, _W)], idxw)
            pltpu.sync_copy(tidx_hbm.at[pl.ds(b, _W)], tidxw)
            pltpu.async_copy(kp_hbm.at[idxw], rows, s1).wait()
            pltpu.async_copy(rows, out_hbm.at[tidxw], s2).wait()

    return body(kpp, idx_flat, tidx_flat)


def _sub_body(n_ref, q_ref, o_ref):
    n3 = n_ref[...].reshape(_QB, 8, _D)
    qb = q_ref[...]
    o_ref[0] = (n3 - qb[:, None, :])[:, :_K, :]


def kernel(x, y):
    q = _extract_patches(x, _P, _S)[0]
    kp = _extract_patches(y, _P, _S)[0]
    pad = _QPAD - _Q
    qp = jnp.pad(q, ((0, pad), (0, 0)))
    kpp = jnp.pad(kp, ((0, pad), (0, 0)))
    kpt = kpp.T

    scores, idxs = pl.pallas_call(
        _topk_body,
        grid=(_NBLK,),
        in_specs=[
            pl.BlockSpec((_QB, _D), lambda i: (i, 0)),
            pl.BlockSpec((_D, _QPAD), lambda i: (0, 0)),
        ],
        out_specs=[
            pl.BlockSpec((_QB, 8), lambda i: (i, 0)),
            pl.BlockSpec((_QB, 8), lambda i: (i, 0)),
        ],
        out_shape=[
            jax.ShapeDtypeStruct((_QPAD, 8), jnp.float32),
            jax.ShapeDtypeStruct((_QPAD, 8), jnp.int32),
        ],
        scratch_shapes=[
            pltpu.VMEM((_D, _QPAD), jnp.bfloat16),
            pltpu.VMEM((8, _QPAD), jnp.float32),
        ],
    )(qp, kpt)

    t = jnp.arange(_TPAD, dtype=jnp.int32)
    real = t < _QPAD * _K
    idx_flat = jnp.pad(idxs[:, :_K].reshape(_QPAD * _K),
                       (0, _TPAD - _QPAD * _K))
    tidx_flat = jnp.where(real, 8 * (t // _K) + t % _K,
                          8 * (t - _QPAD * _K) + 7)

    neigh8 = _sc_gather(kpp, idx_flat, tidx_flat)

    diff_patch = pl.pallas_call(
        _sub_body,
        grid=(_NBLK,),
        in_specs=[
            pl.BlockSpec((_QB * 8, _D), lambda i: (i, 0)),
            pl.BlockSpec((_QB, _D), lambda i: (i, 0)),
        ],
        out_specs=pl.BlockSpec((1, _QB, _K, _D), lambda i: (0, i, 0, 0)),
        out_shape=jax.ShapeDtypeStruct((1, _Q, _K, _D), jnp.float32),
    )(neigh8, qp)

    score_k = scores[:_Q, :_K][None]
    idx_k = idxs[:_Q, :_K][None]
    return (score_k, idx_k, diff_patch)


# speedup vs baseline: 4.7807x; 1.1224x over previous
import functools

import jax
import jax.numpy as jnp
from jax.experimental import pallas as pl
from jax.experimental.pallas import tpu as pltpu
from jax.experimental.pallas import tpu_sc as plsc

_K = 5
_P = 3
_S = 1

_Q = 2116
_D = 2304
_QB = 128
_NBLK = 17
_QPAD = _QB * _NBLK
_NEG = -3.0e38


def _extract_patches(feat, p, s):
    patches = jax.lax.conv_general_dilated_patches(
        feat, filter_shape=(p, p), window_strides=(s, s), padding='VALID')
    B, D, nh, nw = patches.shape
    return patches.reshape(B, D, nh * nw).transpose(0, 2, 1)


def _topk_body(q_ref, kt_ref, sc_ref, ix_ref, kb16_ref, k2_ref):
    i = pl.program_id(0)

    @pl.when(i == 0)
    def _():
        kt = kt_ref[...]
        kb16_ref[...] = kt.astype(jnp.bfloat16)
        k2_ref[0:1, :] = jnp.sum(kt * kt, axis=0, keepdims=True)

    qb = q_ref[...]
    q2 = jnp.sum(qb * qb, axis=1)
    qk = jax.lax.dot_general(
        qb.astype(jnp.bfloat16), kb16_ref[...],
        dimension_numbers=(((1,), (0,)), ((), ())),
        preferred_element_type=jnp.float32)
    t = 2.0 * qk - k2_ref[0:1, :]
    col = jax.lax.broadcasted_iota(jnp.int32, (_QB, _Q), 1)
    for kk in range(_K):
        m = jnp.max(t, axis=1)
        a = jnp.argmax(t, axis=1).astype(jnp.int32)
        sc_ref[:, kk] = m - q2
        ix_ref[:, kk] = a
        t = jnp.where(col == a[:, None], _NEG, t)


_NW = 32
_TPW = 352
_TPAD = _NW * _TPW
_W = 16
_NWIN = _TPW // _W


def _sc_gather(kpp, idx2d, tidx2d):
    mesh = plsc.VectorSubcoreMesh(core_axis_name="c", subcore_axis_name="s")

    @functools.partial(
        pl.kernel, mesh=mesh,
        out_type=jax.ShapeDtypeStruct((_QPAD * 8, _D), jnp.float32),
        scratch_types=[
            pltpu.VMEM((_NWIN, _W), jnp.int32),
            pltpu.VMEM((_NWIN, _W), jnp.int32),
            pltpu.VMEM((_W, _D), jnp.float32),
            pltpu.VMEM((_W, _D), jnp.float32),
            pltpu.SemaphoreType.DMA,
            pltpu.SemaphoreType.DMA,
            pltpu.SemaphoreType.DMA,
            pltpu.SemaphoreType.DMA,
        ],
    )
    def body(kp_hbm, idx_hbm, tidx_hbm, out_hbm, idxv, tidxv,
             rows0, rows1, g0, g1, s0, s1):
        wid = jax.lax.axis_index("s") * 2 + jax.lax.axis_index("c")
        pltpu.sync_copy(idx_hbm.at[wid], idxv)
        pltpu.sync_copy(tidx_hbm.at[wid], tidxv)
        bufs = (rows0, rows1)
        gsems = (g0, g1)
        ssems = (s0, s1)
        pltpu.async_copy(kp_hbm.at[idxv.at[0]], rows0, g0)
        pltpu.async_copy(kp_hbm.at[idxv.at[1]], rows1, g1)

        @pl.loop(0, _NWIN, step=2)
        def _(w0):
            for b in range(2):
                w = w0 + b
                pltpu.make_async_copy(kp_hbm.at[idxv.at[w]],
                                      bufs[b], gsems[b]).wait()
                sc = pltpu.async_copy(bufs[b], out_hbm.at[tidxv.at[w]],
                                      ssems[b])
                sc.wait()

                @pl.when(w + 2 < _NWIN)
                def _():
                    pltpu.async_copy(kp_hbm.at[idxv.at[w + 2]],
                                     bufs[b], gsems[b])

    return body(kpp, idx2d, tidx2d)


def _sub_body(n_ref, q_ref, o_ref):
    n3 = n_ref[...].reshape(_QB, 8, _D)
    qb = q_ref[...]
    o_ref[0] = (n3 - qb[:, None, :])[:, :_K, :]


def kernel(x, y):
    q = _extract_patches(x, _P, _S)[0]
    kp = _extract_patches(y, _P, _S)[0]
    kpt = kp.T

    scores, idxs = pl.pallas_call(
        _topk_body,
        grid=(_NBLK,),
        in_specs=[
            pl.BlockSpec((_QB, _D), lambda i: (i, 0)),
            pl.BlockSpec((_D, _Q), lambda i: (0, 0)),
        ],
        out_specs=[
            pl.BlockSpec((_QB, 8), lambda i: (i, 0)),
            pl.BlockSpec((_QB, 8), lambda i: (i, 0)),
        ],
        out_shape=[
            jax.ShapeDtypeStruct((_QPAD, 8), jnp.float32),
            jax.ShapeDtypeStruct((_QPAD, 8), jnp.int32),
        ],
        scratch_shapes=[
            pltpu.VMEM((_D, _Q), jnp.bfloat16),
            pltpu.VMEM((8, _Q), jnp.float32),
        ],
    )(q, kpt)

    t = jnp.arange(_TPAD, dtype=jnp.int32)
    real = t < _QPAD * _K
    idx_flat = jnp.pad(idxs[:, :_K].reshape(_QPAD * _K),
                       (0, _TPAD - _QPAD * _K))
    tidx_flat = jnp.where(real, 8 * (t // _K) + t % _K,
                          8 * (t - _QPAD * _K) + 7)

    neigh8 = _sc_gather(kp, idx_flat.reshape(_NW, _NWIN, _W),
                        tidx_flat.reshape(_NW, _NWIN, _W))

    diff_patch = pl.pallas_call(
        _sub_body,
        grid=(_NBLK,),
        in_specs=[
            pl.BlockSpec((_QB * 8, _D), lambda i: (i, 0)),
            pl.BlockSpec((_QB, _D), lambda i: (i, 0)),
        ],
        out_specs=pl.BlockSpec((1, _QB, _K, _D), lambda i: (0, i, 0, 0)),
        out_shape=jax.ShapeDtypeStruct((1, _Q, _K, _D), jnp.float32),
    )(neigh8, q)

    score_k = scores[:_Q, :_K][None]
    idx_k = idxs[:_Q, :_K][None]
    return (score_k, idx_k, diff_patch)
